# baseline (speedup 1.0000x reference)
import functools

import jax
import jax.numpy as jnp
import numpy as np
from jax.experimental import pallas as pl
from jax.experimental.pallas import tpu as pltpu


def _dft_mats(C: int):
    H = C // 2
    c = np.arange(C)[:, None].astype(np.float64)
    f = np.arange(H)[None, :].astype(np.float64)
    ang = 2.0 * np.pi * c * f / C
    cosm = np.cos(ang)
    sinm = -np.sin(ang)
    sinm[:, 0] = np.cos(np.pi * np.arange(C))
    fwd = np.concatenate([cosm, sinm], axis=1)
    w = np.full((1, H), 2.0 / C)
    w[0, 0] = 1.0 / C
    icos = np.cos(ang) * w
    isin = -np.sin(ang) * w
    isin[:, 0] = np.cos(np.pi * np.arange(C)) / C
    inv = np.concatenate([icos, isin], axis=1).T
    return fwd.astype(np.float32), inv.astype(np.float32)


def _spec_mag(x, fwdh, fwdl, *, H, FP):
    xh = x.astype(jnp.bfloat16)
    xl = (x - xh.astype(jnp.float32)).astype(jnp.bfloat16)
    spec = (jnp.dot(xh, fwdh, preferred_element_type=jnp.float32)
            + (jnp.dot(xh, fwdl, preferred_element_type=jnp.float32)
               + jnp.dot(xl, fwdh, preferred_element_type=jnp.float32)))
    re = spec[:, :H]
    imt = spec[:, H:]
    lane = jax.lax.broadcasted_iota(jnp.int32, re.shape, 1)
    im = jnp.where(lane == 0, jnp.float32(0.0), imt)
    mag_lo = re * re + im * im
    pad = jax.lax.broadcasted_iota(jnp.int32, (re.shape[0], FP - H), 1)
    neg_inf = jnp.float32(-jnp.inf)
    nyq = jnp.where(pad == 0, imt[:, 0:1] * imt[:, 0:1], neg_inf)
    return spec, jnp.concatenate([mag_lo, nyq], axis=1)


def _tail(x, spec, mag, thr, inv, w1, b1, w2, b2, w3, b3, *, H):
    cols = jax.lax.broadcasted_iota(jnp.int32, mag.shape, 1)
    keep = jnp.where((mag >= thr) & (cols <= H),
                     jnp.float32(1.0), jnp.float32(0.0))
    keepc = jnp.concatenate(
        [keep[:, :H], keep[:, H:H + 1], keep[:, 1:H]], axis=1)

    lo = jax.lax.Precision.DEFAULT
    spec_m = spec * keepc
    x_filt = jnp.dot(spec_m, inv,
                     preferred_element_type=jnp.float32, precision=lo)
    pf = jnp.maximum(
        jnp.dot(x_filt, w1, preferred_element_type=jnp.float32,
                precision=lo) + b1, 0.0)
    comb = jnp.concatenate([pf, x], axis=1)
    h = jnp.maximum(
        jnp.dot(comb, w2, preferred_element_type=jnp.float32,
                precision=lo) + b2, 0.0)
    out_mlp = jnp.dot(h, w3, preferred_element_type=jnp.float32,
                      precision=lo) + b3
    return (x - x_filt) + out_mlp


def _fan_block(x_ref, fwdh_ref, fwdl_ref, inv_ref, w1_ref, b1_ref, w2_ref,
               b2_ref, w3_ref, b3_ref, o_ref, *, H: int, FP: int, K: int):
    TB = x_ref.shape[0]
    neg_inf = jnp.float32(-jnp.inf)
    args = (inv_ref[...], w1_ref[...], b1_ref[...], w2_ref[...],
            b2_ref[...], w3_ref[...], b3_ref[...])
    fh = fwdh_ref[...]
    fl = fwdl_ref[...]
    NW = 4
    TQ = TB // NW
    xs = [x_ref[i * TQ:(i + 1) * TQ, :] for i in range(NW)]
    sm = [_spec_mag(xq, fh, fl, H=H, FP=FP) for xq in xs]
    specs = [s for s, _ in sm]
    ms = [m for _, m in sm]
    mxs = [jnp.max(m, axis=1, keepdims=True) for m in ms]
    for _ in range(K - 1):
        mxs = [jnp.max(jnp.where(m >= mx, neg_inf, m), axis=1, keepdims=True)
               for m, mx in zip(ms, mxs)]
    for i in range(NW):
        o_ref[i * TQ:(i + 1) * TQ, :] = _tail(xs[i], specs[i], ms[i], mxs[i],
                                              *args, H=H)


@jax.jit
def kernel(x, W1, b1, W2, b2, W3, b3):
    B, S, C = x.shape
    H = C // 2
    FP = H + 128
    K = min(20, H + 1)
    T = B * S
    TB = 1024 if T % 1024 == 0 else T

    fwd_np, inv_np = _dft_mats(C)
    fwd = jnp.asarray(fwd_np)
    fwd_h = fwd.astype(jnp.bfloat16)
    fwd_l = (fwd - fwd_h.astype(jnp.float32)).astype(jnp.bfloat16)
    inv = jnp.asarray(inv_np)

    H1 = W1.shape[1]
    H1P = 128
    w1p = jnp.zeros((C, H1P), jnp.float32).at[:, :H1].set(W1)
    b1p = jnp.zeros((1, H1P), jnp.float32).at[0, :H1].set(b1)
    H2 = W2.shape[1]
    w2p = jnp.zeros((H1P + C, H2), jnp.float32)
    w2p = w2p.at[:H1, :].set(W2[:H1, :]).at[H1P:, :].set(W2[H1:, :])
    b2r = b2.reshape(1, H2)
    b3r = b3.reshape(1, C)

    xt = x.reshape(T, C)
    full = lambda shape: pl.BlockSpec(shape, lambda i: (0, 0))
    out = pl.pallas_call(
        functools.partial(_fan_block, H=H, FP=FP, K=K),
        grid=(T // TB,),
        in_specs=[
            pl.BlockSpec((TB, C), lambda i: (i, 0)),
            full((C, C)),
            full((C, C)),
            full((C, C)),
            full((C, H1P)),
            full((1, H1P)),
            full((H1P + C, H2)),
            full((1, H2)),
            full((H2, C)),
            full((1, C)),
        ],
        out_specs=pl.BlockSpec((TB, C), lambda i: (i, 0)),
        out_shape=jax.ShapeDtypeStruct((T, C), jnp.float32),
        compiler_params=pltpu.CompilerParams(
            dimension_semantics=("parallel",)),
    )(xt, fwd_h, fwd_l, inv, w1p, b1p, w2p, b2r, W3, b3r)
    return out.reshape(B, S, C)
